# baseline (speedup 1.0000x reference)
import jax
import jax.numpy as jnp
from jax import lax
from jax.experimental import pallas as pl
from jax.experimental.pallas import tpu as pltpu
from jax.experimental.pallas import tpu_sc as plsc

N = 100000
D = 128
G = 1024

NC = 2
NS = 16
NW = NC * NS

L = 16
NV = D // L
CHUNK = 128
NGRP = CHUNK // L
FULL = N // CHUNK
TAIL = N - FULL * CHUNK
BASE = FULL // NW
EXTRA = FULL - BASE * NW
NBUF = 2
OUTER = (BASE + 1 + NBUF - 1) // NBUF

STAGE = 48
FLUSH_AT = 32

ROWS_PER_TILE = G // NS
JUNK = G


def _seg_body(h_hbm, batch_hbm, out_hbm,
              hb0, hb1, ib0, ib1,
              tbuf, tib0, tib1, zbuf, stage, istage, acc,
              ps0, ps1):
    cid = lax.axis_index("c")
    sid = lax.axis_index("s")
    wid = sid * NC + cid
    HB = (hb0, hb1)
    IB = (ib0, ib1)
    PS = (ps0, ps1)

    lanes = lax.iota(jnp.int32, L)
    zvec = jnp.zeros((L,), jnp.float32)

    def zrow(r, carry):
        for c in range(NV):
            zbuf[r, pl.ds(c * L, L)] = zvec
        return carry

    lax.fori_loop(0, ROWS_PER_TILE, zrow, 0)
    pltpu.sync_copy(zbuf, acc.at[pl.ds(sid * ROWS_PER_TILE, ROWS_PER_TILE)])
    for s in range(STAGE // L):
        istage[pl.ds(s * L, L)] = jnp.full((L,), JUNK, jnp.int32)
    plsc.subcore_barrier()

    c0 = BASE * wid + jnp.minimum(wid, EXTRA)
    cnt_chunks = BASE + (wid < EXTRA).astype(jnp.int32)

    def prefetch(b, k):
        off = c0 + k
        pltpu.async_copy(h_hbm.at[pl.ds(off * CHUNK, CHUNK)], HB[b], PS[b])
        pltpu.async_copy(batch_hbm.at[pl.ds(off * NGRP, NGRP)], IB[b], PS[b])

    for b in range(NBUF):
        prefetch(b, b)

    def flush(c):
        pltpu.sync_copy(stage, acc.at[istage], add=True)
        for s in range(STAGE // L):
            istage[pl.ds(s * L, L)] = jnp.full((L,), JUNK, jnp.int32)
        return jnp.int32(0)

    def put(accs, seg, cnt):
        for c in range(NV):
            stage[cnt, pl.ds(c * L, L)] = accs[c]
        iv = istage[pl.ds(0, L)]
        istage[pl.ds(0, L)] = jnp.where(lanes == cnt, seg, iv)
        iv = istage[pl.ds(L, L)]
        istage[pl.ds(L, L)] = jnp.where(lanes == cnt - L, seg, iv)
        iv = istage[pl.ds(2 * L, L)]
        istage[pl.ds(2 * L, L)] = jnp.where(lanes == cnt - 2 * L, seg, iv)

    def process_group(hb, ib, g, cnt):
        v = ib[g]
        vl0 = v[0]
        vl15 = v[L - 1]
        any_start = v[1] != vl0
        for l in range(2, L):
            any_start = any_start | (v[l] != v[l - 1])

        def load_row(r):
            return tuple(hb[r, pl.ds(c * L, L)] for c in range(NV))

        def fast(cnt):
            accs = load_row(g * L)
            for l in range(1, L):
                row = load_row(g * L + l)
                accs = tuple(accs[c] + row[c] for c in range(NV))
            put(accs, vl0, cnt)
            return cnt + 1

        def mixed(cnt):
            pltpu.sync_copy(hb.at[pl.ds(g * L, L)], acc.at[ib.at[g]],
                            add=True)
            return cnt

        return lax.cond(any_start, mixed, fast, cnt)

    def outer(j, cnt):
        for b in range(NBUF):
            k = j * NBUF + b

            def do(cnt, b=b, k=k):
                pltpu.make_async_copy(
                    h_hbm.at[pl.ds(0, CHUNK)], HB[b], PS[b]).wait()
                pltpu.make_async_copy(
                    batch_hbm.at[pl.ds(0, NGRP)], IB[b], PS[b]).wait()

                for g in range(NGRP):
                    cnt = process_group(HB[b], IB[b], g, cnt)

                cnt = lax.cond(cnt >= FLUSH_AT, flush, lambda c: c, cnt)

                @pl.when(k + NBUF < cnt_chunks)
                def _():
                    prefetch(b, k + NBUF)

                return cnt

            cnt = lax.cond(k < cnt_chunks, do, lambda c: c, cnt)
        return cnt

    cnt = lax.fori_loop(0, OUTER, outer, jnp.int32(0))

    @pl.when(cnt > 0)
    def _():
        flush(cnt)

    @pl.when(wid == NW - 1)
    def _():
        pltpu.sync_copy(h_hbm.at[pl.ds(FULL * CHUNK, TAIL)], tbuf)
        pltpu.sync_copy(batch_hbm.at[FULL * NGRP], tib0)
        pltpu.sync_copy(batch_hbm.at[FULL * NGRP + 1], tib1)
        pltpu.sync_copy(tbuf.at[pl.ds(0, L)], acc.at[tib0], add=True)
        pltpu.sync_copy(tbuf.at[pl.ds(L, L)], acc.at[tib1], add=True)

    plsc.subcore_barrier()

    pltpu.sync_copy(
        acc.at[pl.ds(sid * ROWS_PER_TILE, ROWS_PER_TILE)],
        out_hbm.at[cid, pl.ds(sid * ROWS_PER_TILE, ROWS_PER_TILE)],
    )


def _segment_sum_sc(h, batch2):
    mesh = plsc.VectorSubcoreMesh(core_axis_name="c", subcore_axis_name="s")
    return pl.kernel(
        _seg_body,
        out_type=jax.ShapeDtypeStruct((NC, G, D), jnp.float32),
        mesh=mesh,
        scratch_types=[
            pltpu.VMEM((CHUNK, D), jnp.float32),
            pltpu.VMEM((CHUNK, D), jnp.float32),
            pltpu.VMEM((NGRP, L), jnp.int32),
            pltpu.VMEM((NGRP, L), jnp.int32),
            pltpu.VMEM((TAIL, D), jnp.float32),
            pltpu.VMEM((L,), jnp.int32),
            pltpu.VMEM((L,), jnp.int32),
            pltpu.VMEM((ROWS_PER_TILE, D), jnp.float32),
            pltpu.VMEM((STAGE, D), jnp.float32),
            pltpu.VMEM((STAGE,), jnp.int32),
            pltpu.VMEM_SHARED((G + 8, D), jnp.float32),
            pltpu.SemaphoreType.DMA,
            pltpu.SemaphoreType.DMA,
        ],
    )(h, batch2)


def _mlp_body(p_ref, v_ref, w_ref, b_ref, o_ref):
    pooled = p_ref[0] + p_ref[1]
    o_ref[...] = (
        v_ref[...]
        + lax.dot_general(pooled, w_ref[...], (((1,), (1,)), ((), ())),
                          preferred_element_type=jnp.float32)
        + b_ref[...]
    )


def _mlp_tc(part, v, w, b2):
    return pl.pallas_call(
        _mlp_body,
        out_shape=jax.ShapeDtypeStruct((G, D), jnp.float32),
    )(part, v, w, b2)


def kernel(h, batch, v, W, b):
    batch2 = batch.astype(jnp.int32).reshape(N // L, L)
    part = _segment_sum_sc(h, batch2)
    return _mlp_tc(part, v, W, b.reshape(1, D))


# speedup vs baseline: 2.5022x; 2.5022x over previous
import jax
import jax.numpy as jnp
from jax import lax
from jax.experimental import pallas as pl
from jax.experimental.pallas import tpu as pltpu
from jax.experimental.pallas import tpu_sc as plsc

N = 100000
D = 128
G = 1024

NC = 2
NS = 16
NW = NC * NS

CHUNK = 128
FULL = N // CHUNK
TAIL = N - FULL * CHUNK
BASE = FULL // NW
EXTRA = FULL - BASE * NW
NBUF = 4
OUTER = (BASE + 1 + NBUF - 1) // NBUF

ROWS_PER_TILE = G // NS


def _seg_body(h_hbm, batch_hbm, out_hbm, hb0, hb1, hb2, hb3,
              ib0, ib1, ib2, ib3,
              tbuf, tibuf, zbuf, acc, ps0, ps1, ps2, ps3):
    cid = lax.axis_index("c")
    sid = lax.axis_index("s")
    wid = sid * NC + cid
    HB = (hb0, hb1, hb2, hb3)
    IB = (ib0, ib1, ib2, ib3)
    PS = (ps0, ps1, ps2, ps3)

    def zrow(r, carry):
        for c8 in range(D // 16):
            zbuf[r, pl.ds(c8 * 16, 16)] = jnp.zeros((16,), jnp.float32)
        return carry

    lax.fori_loop(0, ROWS_PER_TILE, zrow, 0)
    pltpu.sync_copy(zbuf, acc.at[pl.ds(sid * ROWS_PER_TILE, ROWS_PER_TILE)])
    plsc.subcore_barrier()

    c0 = BASE * wid + jnp.minimum(wid, EXTRA)
    cnt = BASE + (wid < EXTRA).astype(jnp.int32)

    def prefetch(b, k):
        off = (c0 + k) * CHUNK
        pltpu.async_copy(h_hbm.at[pl.ds(off, CHUNK)], HB[b], PS[b])
        pltpu.async_copy(batch_hbm.at[pl.ds(off, CHUNK)], IB[b], PS[b])

    for b in range(NBUF):
        prefetch(b, b)

    def outer(j, carry):
        for b in range(NBUF):
            k = j * NBUF + b

            @pl.when(k < cnt)
            def _(b=b, k=k):
                pltpu.make_async_copy(
                    h_hbm.at[pl.ds(0, CHUNK)], HB[b], PS[b]).wait()
                pltpu.make_async_copy(
                    batch_hbm.at[pl.ds(0, CHUNK)], IB[b], PS[b]).wait()
                pltpu.sync_copy(HB[b], acc.at[IB[b]], add=True)

                @pl.when(k + NBUF < cnt)
                def _():
                    prefetch(b, k + NBUF)

        return carry

    lax.fori_loop(0, OUTER, outer, 0)

    @pl.when(wid == NW - 1)
    def _():
        pltpu.sync_copy(h_hbm.at[pl.ds(FULL * CHUNK, TAIL)], tbuf)
        pltpu.sync_copy(batch_hbm.at[pl.ds(FULL * CHUNK, TAIL)], tibuf)
        pltpu.sync_copy(tbuf, acc.at[tibuf], add=True)

    plsc.subcore_barrier()

    pltpu.sync_copy(
        acc.at[pl.ds(sid * ROWS_PER_TILE, ROWS_PER_TILE)],
        out_hbm.at[cid, pl.ds(sid * ROWS_PER_TILE, ROWS_PER_TILE)],
    )


def _segment_sum_sc(h, batch):
    mesh = plsc.VectorSubcoreMesh(core_axis_name="c", subcore_axis_name="s")
    return pl.kernel(
        _seg_body,
        out_type=jax.ShapeDtypeStruct((NC, G, D), jnp.float32),
        mesh=mesh,
        scratch_types=[
            pltpu.VMEM((CHUNK, D), jnp.float32),
            pltpu.VMEM((CHUNK, D), jnp.float32),
            pltpu.VMEM((CHUNK, D), jnp.float32),
            pltpu.VMEM((CHUNK, D), jnp.float32),
            pltpu.VMEM((CHUNK,), jnp.int32),
            pltpu.VMEM((CHUNK,), jnp.int32),
            pltpu.VMEM((CHUNK,), jnp.int32),
            pltpu.VMEM((CHUNK,), jnp.int32),
            pltpu.VMEM((TAIL, D), jnp.float32),
            pltpu.VMEM((TAIL,), jnp.int32),
            pltpu.VMEM((ROWS_PER_TILE, D), jnp.float32),
            pltpu.VMEM_SHARED((G, D), jnp.float32),
            pltpu.SemaphoreType.DMA,
            pltpu.SemaphoreType.DMA,
            pltpu.SemaphoreType.DMA,
            pltpu.SemaphoreType.DMA,
        ],
    )(h, batch)


def _mlp_body(p_ref, v_ref, wt_ref, b_ref, o_ref):
    pooled = p_ref[0] + p_ref[1]
    o_ref[...] = (
        v_ref[...]
        + jnp.dot(pooled, wt_ref[...], preferred_element_type=jnp.float32)
        + b_ref[...]
    )


def _mlp_tc(part, v, w_t, b2):
    return pl.pallas_call(
        _mlp_body,
        out_shape=jax.ShapeDtypeStruct((G, D), jnp.float32),
    )(part, v, w_t, b2)


def kernel(h, batch, v, W, b):
    part = _segment_sum_sc(h, batch.astype(jnp.int32))
    return _mlp_tc(part, v, W.T, b.reshape(1, D))
